# baseline (speedup 1.0000x reference)
import functools

import jax
import jax.numpy as jnp
from jax import lax
from jax.experimental import pallas as pl
from jax.experimental.pallas import tpu as pltpu
from jax.experimental.pallas import tpu_sc as plsc

N = 10000
D = 128
E = 320000

NC = 2
NS = 16
NW = NC * NS
CHUNK = 128
CPT = 80
HALF = CPT // 2
E_PAD = NW * CPT * CHUNK
NBUF = 2
N_PAD = 10112
RPT = N_PAD // NS
RPT_PIECES = (128, 128, 128, 128, 120)


def _sc_aggregate(x, src2, dst2):
    mesh = plsc.VectorSubcoreMesh(core_axis_name="c", subcore_axis_name="s")

    @functools.partial(
        pl.kernel,
        out_type=jax.ShapeDtypeStruct((NC, N_PAD, D), jnp.float32),
        mesh=mesh,
        scratch_types=[
            pltpu.VMEM((2, HALF, CHUNK), jnp.int32),
            pltpu.VMEM((HALF, CHUNK), jnp.int32),
            pltpu.VMEM((NBUF, CHUNK, D), jnp.float32),
            pltpu.VMEM_SHARED((N_PAD, D), jnp.float32),
            pltpu.SemaphoreType.DMA,
            pltpu.SemaphoreType.DMA,
        ],
    )
    def body(x_hbm, src_hbm, dst_hbm, out_hbm, src_v, dst_v, rows_v, acc,
             sem0, sem1):
        c = lax.axis_index("c")
        s = lax.axis_index("s")
        sems = (sem0, sem1)

        crow = (c * NS + s) * CPT
        pltpu.sync_copy(src_hbm.at[pl.ds(crow, HALF)], src_v.at[0])
        pltpu.sync_copy(dst_hbm.at[pl.ds(crow, HALF)], dst_v)

        z = jnp.zeros((16,), jnp.float32)

        def zero_row(i, carry):
            for k in range(D // 16):
                rows_v[0, i, pl.ds(k * 16, 16)] = z
            return carry

        lax.fori_loop(0, CHUNK, zero_row, 0)
        row0 = s * RPT
        off = 0
        for sz in RPT_PIECES:
            pltpu.sync_copy(rows_v.at[0, pl.ds(0, sz)],
                            acc.at[pl.ds(row0 + off, sz)])
            off += sz
        plsc.subcore_barrier()

        def gather(j, b):
            return pltpu.async_copy(
                x_hbm.at[src_v.at[lax.div(j, HALF), lax.rem(j, HALF)]],
                rows_v.at[b], sems[b])

        for b in range(NBUF):
            gather(jnp.int32(b), b)

        def step(t, carry):
            for b in range(NBUF):
                j = t * NBUF + b
                pltpu.make_async_copy(
                    x_hbm.at[src_v.at[lax.div(j, HALF), lax.rem(j, HALF)]],
                    rows_v.at[b], sems[b]).wait()

                @pl.when(j == HALF)
                def _():
                    pltpu.sync_copy(dst_hbm.at[pl.ds(crow + HALF, HALF)],
                                    dst_v)

                pltpu.sync_copy(rows_v.at[b], acc.at[dst_v.at[lax.rem(j, HALF)]],
                                add=True)

                @pl.when(j + NBUF == HALF)
                def _():
                    pltpu.sync_copy(src_hbm.at[pl.ds(crow + HALF, HALF)],
                                    src_v.at[1])

                @pl.when(j + NBUF < CPT)
                def _():
                    gather(j + NBUF, b)
            return carry

        lax.fori_loop(0, CPT // NBUF, step, 0)

        plsc.subcore_barrier()

        off = 0
        for sz in RPT_PIECES:
            pltpu.sync_copy(acc.at[pl.ds(row0 + off, sz)],
                            rows_v.at[0, pl.ds(0, sz)])
            pltpu.sync_copy(rows_v.at[0, pl.ds(0, sz)],
                            out_hbm.at[c, pl.ds(row0 + off, sz)])
            off += sz

    return body(x, src2, dst2)


def _mlp_body(x_ref, agg_ref, eps_ref, w1_ref, b1_ref, w2_ref, b2_ref,
              out_ref):
    h = x_ref[...] * eps_ref[0, 0] + agg_ref[0] + agg_ref[1]
    h = jnp.dot(h, w1_ref[...], preferred_element_type=jnp.float32,
                precision=lax.Precision.HIGHEST) + b1_ref[...]
    h = jnp.maximum(h, 0.0)
    out_ref[...] = jnp.dot(h, w2_ref[...], preferred_element_type=jnp.float32,
                           precision=lax.Precision.HIGHEST) + b2_ref[...]


def _mlp(x, agg2, eps, W1, b1, W2, b2):
    BR = 1000
    return pl.pallas_call(
        _mlp_body,
        grid=(N // BR,),
        in_specs=[
            pl.BlockSpec((BR, D), lambda i: (i, 0)),
            pl.BlockSpec((NC, BR, D), lambda i: (0, i, 0)),
            pl.BlockSpec(memory_space=pltpu.SMEM),
            pl.BlockSpec((D, D), lambda i: (0, 0)),
            pl.BlockSpec((1, D), lambda i: (0, 0)),
            pl.BlockSpec((D, D), lambda i: (0, 0)),
            pl.BlockSpec((1, D), lambda i: (0, 0)),
        ],
        out_specs=pl.BlockSpec((BR, D), lambda i: (i, 0)),
        out_shape=jax.ShapeDtypeStruct((N, D), jnp.float32),
    )(x, agg2, eps.reshape(1, 1), W1, b1.reshape(1, D), W2, b2.reshape(1, D))


def kernel(x, edge_index, eps, W1, b1, W2, b2):
    npad = E_PAD - E
    pad_src = jnp.zeros((npad,), jnp.int32)
    pad_dst = N + (jnp.arange(npad, dtype=jnp.int32) % (N_PAD - N))
    src2 = jnp.concatenate([edge_index[0], pad_src]).reshape(-1, CHUNK)
    dst2 = jnp.concatenate([edge_index[1], pad_dst]).reshape(-1, CHUNK)
    agg2 = _sc_aggregate(x, src2, dst2)
    return _mlp(x, agg2, eps, W1, b1, W2, b2)
